# Initial kernel scaffold; baseline (speedup 1.0000x reference)
import functools
import math

import jax
import jax.numpy as jnp
from jax import lax
from jax.experimental import pallas as pl
from jax.experimental.pallas import tpu as pltpu
from jax.experimental.pallas import tpu_sc as plsc

_WINDOW = 64
_CAP = 4096
_KS = 13
_NEG = jnp.finfo(jnp.float32).min


def _pooled_prologue(q, k):
    B, H, S, D = q.shape
    q_win = q[:, :, -_WINDOW:, :]
    attn = jnp.einsum('bhqd,bhkd->bhqk', q_win, k) / math.sqrt(D)
    idx = jnp.arange(_WINDOW)
    causal = jnp.where(idx[:, None] >= idx[None, :], 0.0, _NEG)
    attn = attn.at[:, :, :, -_WINDOW:].add(causal[None, None, :, :])
    attn = jax.nn.softmax(attn.astype(jnp.float32), axis=-1)
    attn_sum = attn[:, :, :, : S - _WINDOW].sum(axis=-2)
    pad = _KS // 2
    pooled = jax.lax.reduce_window(
        attn_sum, 0.0, jax.lax.add,
        window_dimensions=(1, 1, _KS),
        window_strides=(1, 1, 1),
        padding=((0, 0), (0, 0), (pad, pad)),
    ) / float(_KS)
    return pooled


def _rank_body(vrow_ref, ranks_ref):
    S = vrow_ref.shape[1]
    vrow = vrow_ref[...]
    vcol = jnp.reshape(vrow, (S, 1))
    jidx = lax.broadcasted_iota(jnp.int32, (S, 128), 0)
    lidx = lax.broadcasted_iota(jnp.int32, (S, 128), 1)
    jml = jidx - lidx

    def body(ic, _):
        vi = vrow_ref[0:1, pl.ds(pl.multiple_of(ic * 128, 128), 128)]
        gt = vcol > vi
        eq = vcol == vi
        jlt = jml < ic * 128
        cnt = jnp.sum((gt | (eq & jlt)).astype(jnp.int32), axis=0,
                      keepdims=True)
        ranks_ref[0:1, pl.ds(pl.multiple_of(ic * 128, 128), 128)] = cnt
        return 0

    lax.fori_loop(0, S // 128, body, 0)


def _ranks_call(pooled_pad):
    BH, S = pooled_pad.shape
    return pl.pallas_call(
        _rank_body,
        grid=(BH,),
        in_specs=[pl.BlockSpec((1, S), lambda h: (h, 0))],
        out_specs=pl.BlockSpec((1, S), lambda h: (h, 0)),
        out_shape=jax.ShapeDtypeStruct((BH, S), jnp.int32),
    )(pooled_pad)


def _sc_gather(k2, v2, ranks, BH, S, D, topk):
    cap = topk + _WINDOW
    nchunk = cap // 128
    mesh = plsc.VectorSubcoreMesh(core_axis_name="c", subcore_axis_name="s")
    num_cores = mesh.num_cores

    @functools.partial(
        pl.kernel,
        out_type=[jax.ShapeDtypeStruct((BH * cap, D), jnp.float32),
                  jax.ShapeDtypeStruct((BH * cap, D), jnp.float32)],
        mesh=mesh,
        scratch_types=[
            pltpu.VMEM((S,), jnp.int32),
            pltpu.VMEM((nchunk, 128), jnp.int32),
            pltpu.VMEM((128, D), jnp.float32),
            pltpu.VMEM((128, D), jnp.float32),
            pltpu.SemaphoreType.DMA,
            pltpu.SemaphoreType.DMA,
        ],
    )
    def sc_k(k2_hbm, v2_hbm, ranks_hbm, ko_hbm, vo_hbm,
             rank_v, idx2, bufk, bufv, semk, semv):
        w = lax.axis_index("s") * num_cores + lax.axis_index("c")
        pltpu.sync_copy(ranks_hbm.at[w], rank_v)

        def build(c, _):
            r = rank_v[pl.ds(c * 16, 16)]
            src = w * S + c * 16 + lax.iota(jnp.int32, 16)
            rhi = lax.shift_right_logical(r, 7)
            rlo = lax.bitwise_and(r, 127)
            plsc.store_scatter(idx2, [rhi, rlo], src, mask=r < topk)
            return 0

        lax.fori_loop(0, (S - _WINDOW) // 16, build, 0)

        for t in range(_WINDOW // 16):
            pos = topk + t * 16
            idx2[pos // 128, pl.ds(pos % 128, 16)] = (
                w * S + (S - _WINDOW) + t * 16 + lax.iota(jnp.int32, 16))

        def chunk(c, _):
            ck = pltpu.async_copy(k2_hbm.at[idx2.at[c]], bufk, semk)
            cv = pltpu.async_copy(v2_hbm.at[idx2.at[c]], bufv, semv)
            ck.wait()
            pltpu.sync_copy(bufk, ko_hbm.at[pl.ds(w * cap + c * 128, 128)])
            cv.wait()
            pltpu.sync_copy(bufv, vo_hbm.at[pl.ds(w * cap + c * 128, 128)])
            return 0

        lax.fori_loop(0, nchunk, chunk, 0)

    return sc_k(k2, v2, ranks)


def kernel(q, k, v):
    B, H, S, D = q.shape
    BH = B * H
    topk = _CAP - _WINDOW
    pooled = _pooled_prologue(q, k)
    pooled_pad = jnp.concatenate(
        [pooled.reshape(BH, S - _WINDOW),
         jnp.full((BH, _WINDOW), -1.0, jnp.float32)], axis=1)
    ranks = _ranks_call(pooled_pad)
    k2 = k.reshape(BH * S, D)
    v2 = v.reshape(BH * S, D)
    ko, vo = _sc_gather(k2, v2, ranks, BH, S, D, topk)
    key_states = ko.reshape(B, H, _CAP, D)
    value_states = vo.reshape(B, H, _CAP, D)
    return (key_states, value_states)


# speedup vs baseline: 1.1069x; 1.1069x over previous
import functools
import math

import jax
import jax.numpy as jnp
from jax import lax
from jax.experimental import pallas as pl
from jax.experimental.pallas import tpu as pltpu
from jax.experimental.pallas import tpu_sc as plsc

_WINDOW = 64
_CAP = 4096
_KS = 13
_NEG = jnp.finfo(jnp.float32).min


def _pooled_prologue(q, k):
    B, H, S, D = q.shape
    q_win = q[:, :, -_WINDOW:, :]
    attn = jnp.einsum('bhqd,bhkd->bhqk', q_win, k) / math.sqrt(D)
    idx = jnp.arange(_WINDOW)
    causal = jnp.where(idx[:, None] >= idx[None, :], 0.0, _NEG)
    attn = attn.at[:, :, :, -_WINDOW:].add(causal[None, None, :, :])
    attn = jax.nn.softmax(attn.astype(jnp.float32), axis=-1)
    attn_sum = attn[:, :, :, : S - _WINDOW].sum(axis=-2)
    pad = _KS // 2
    pooled = jax.lax.reduce_window(
        attn_sum, 0.0, jax.lax.add,
        window_dimensions=(1, 1, _KS),
        window_strides=(1, 1, 1),
        padding=((0, 0), (0, 0), (pad, pad)),
    ) / float(_KS)
    return pooled


def _rank_body(vrow_ref, ranks_ref):
    S = vrow_ref.shape[2]
    vrow = vrow_ref[0]
    vcol = jnp.reshape(vrow, (S, 1))
    jidx = lax.broadcasted_iota(jnp.int32, (S, 128), 0)
    lidx = lax.broadcasted_iota(jnp.int32, (S, 128), 1)
    jml = jidx - lidx

    def body(ic, _):
        vi = vrow_ref[0, 0:1, pl.ds(pl.multiple_of(ic * 128, 128), 128)]
        gt = vcol > vi
        eq = vcol == vi
        jlt = jml < ic * 128
        cnt = jnp.sum((gt | (eq & jlt)).astype(jnp.int32), axis=0,
                      keepdims=True)
        ranks_ref[0, 0:1, pl.ds(pl.multiple_of(ic * 128, 128), 128)] = cnt
        return 0

    lax.fori_loop(0, S // 128, body, 0)


def _ranks_call(pooled_pad):
    BH, S = pooled_pad.shape
    p3 = pooled_pad.reshape(BH, 1, S)
    out = pl.pallas_call(
        _rank_body,
        grid=(BH,),
        in_specs=[pl.BlockSpec((1, 1, S), lambda h: (h, 0, 0))],
        out_specs=pl.BlockSpec((1, 1, S), lambda h: (h, 0, 0)),
        out_shape=jax.ShapeDtypeStruct((BH, 1, S), jnp.int32),
    )(p3)
    return out.reshape(BH, S)


def _sc_gather(k2, v2, ranks, BH, S, D, topk):
    cap = topk + _WINDOW
    nchunk = cap // 128
    mesh = plsc.VectorSubcoreMesh(core_axis_name="c", subcore_axis_name="s")
    num_cores = mesh.num_cores

    @functools.partial(
        pl.kernel,
        out_type=[jax.ShapeDtypeStruct((BH * cap, D), jnp.float32),
                  jax.ShapeDtypeStruct((BH * cap, D), jnp.float32)],
        mesh=mesh,
        scratch_types=[
            pltpu.VMEM((S,), jnp.int32),
            pltpu.VMEM((cap,), jnp.int32),
            pltpu.VMEM((128, D), jnp.float32),
            pltpu.VMEM((128, D), jnp.float32),
            pltpu.SemaphoreType.DMA,
            pltpu.SemaphoreType.DMA,
        ],
        compiler_params=pltpu.CompilerParams(needs_layout_passes=False),
    )
    def sc_k(k2_hbm, v2_hbm, ranks_hbm, ko_hbm, vo_hbm,
             rank_v, idx1, bufk, bufv, semk, semv):
        w = lax.axis_index("s") * num_cores + lax.axis_index("c")
        pltpu.sync_copy(ranks_hbm.at[w], rank_v)

        def build(c, _):
            r = rank_v[pl.ds(c * 16, 16)]
            src = w * S + c * 16 + lax.iota(jnp.int32, 16)
            plsc.store_scatter(idx1, [r], src, mask=r < topk)
            return 0

        lax.fori_loop(0, (S - _WINDOW) // 16, build, 0)

        for t in range(_WINDOW // 16):
            idx1[pl.ds(topk + t * 16, 16)] = (
                w * S + (S - _WINDOW) + t * 16 + lax.iota(jnp.int32, 16))

        def chunk(c, _):
            ix = idx1.at[pl.ds(c * 128, 128)]
            ck = pltpu.async_copy(k2_hbm.at[ix], bufk, semk)
            cv = pltpu.async_copy(v2_hbm.at[ix], bufv, semv)
            ck.wait()
            pltpu.sync_copy(bufk, ko_hbm.at[pl.ds(w * cap + c * 128, 128)])
            cv.wait()
            pltpu.sync_copy(bufv, vo_hbm.at[pl.ds(w * cap + c * 128, 128)])
            return 0

        lax.fori_loop(0, nchunk, chunk, 0)

    return sc_k(k2, v2, ranks)


def kernel(q, k, v):
    B, H, S, D = q.shape
    BH = B * H
    topk = _CAP - _WINDOW
    pooled = _pooled_prologue(q, k)
    pooled_pad = jnp.concatenate(
        [pooled.reshape(BH, S - _WINDOW),
         jnp.full((BH, _WINDOW), -1.0, jnp.float32)], axis=1)
    ranks = _ranks_call(pooled_pad)
    k2 = k.reshape(BH * S, D)
    v2 = v.reshape(BH * S, D)
    ko, vo = _sc_gather(k2, v2, ranks, BH, S, D, topk)
    key_states = ko.reshape(B, H, _CAP, D)
    value_states = vo.reshape(B, H, _CAP, D)
    return (key_states, value_states)


# speedup vs baseline: 1.1564x; 1.0447x over previous
import functools
import math

import jax
import jax.numpy as jnp
from jax import lax
from jax.experimental import pallas as pl
from jax.experimental.pallas import tpu as pltpu
from jax.experimental.pallas import tpu_sc as plsc

_WINDOW = 64
_CAP = 4096
_KS = 13
_NEG = jnp.finfo(jnp.float32).min


def _pooled_prologue(q, k):
    B, H, S, D = q.shape
    q_win = q[:, :, -_WINDOW:, :]
    attn = jnp.einsum('bhqd,bhkd->bhqk', q_win, k) / math.sqrt(D)
    idx = jnp.arange(_WINDOW)
    causal = jnp.where(idx[:, None] >= idx[None, :], 0.0, _NEG)
    attn = attn.at[:, :, :, -_WINDOW:].add(causal[None, None, :, :])
    attn = jax.nn.softmax(attn.astype(jnp.float32), axis=-1)
    attn_sum = attn[:, :, :, : S - _WINDOW].sum(axis=-2)
    pad = _KS // 2
    pooled = jax.lax.reduce_window(
        attn_sum, 0.0, jax.lax.add,
        window_dimensions=(1, 1, _KS),
        window_strides=(1, 1, 1),
        padding=((0, 0), (0, 0), (pad, pad)),
    ) / float(_KS)
    return pooled


def _pooled_body(qw_ref, k_ref, out_ref):
    S = k_ref.shape[1]
    SP = S - _WINDOW
    qw = qw_ref[0]
    kk = k_ref[0]
    scores = lax.dot_general(qw, kk, (((1,), (1,)), ((), ())),
                             preferred_element_type=jnp.float32)
    scores = scores / math.sqrt(qw.shape[1])
    iq = lax.broadcasted_iota(jnp.int32, (_WINDOW, _WINDOW), 0)
    jw = lax.broadcasted_iota(jnp.int32, (_WINDOW, _WINDOW), 1)
    causal = jnp.where(iq >= jw, 0.0, _NEG)
    scores = jnp.concatenate([scores[:, :SP], scores[:, SP:] + causal], axis=1)
    m = jnp.max(scores, axis=1, keepdims=True)
    e = jnp.exp(scores - m)
    s = jnp.sum(e, axis=1, keepdims=True)
    p = e / s
    asum = jnp.sum(p[:, :SP], axis=0, keepdims=True)
    z = jnp.zeros((1, _KS // 2), jnp.float32)
    padded = jnp.concatenate([z, asum, z], axis=1)
    acc = padded[:, 0:SP]
    for u in range(1, _KS):
        acc = acc + padded[:, u:u + SP]
    pooled = acc / float(_KS)
    out_ref[0] = jnp.concatenate(
        [pooled, jnp.full((1, _WINDOW), -1.0, jnp.float32)], axis=1)


def _pooled_call(q, k):
    B, H, S, D = q.shape
    BH = B * H
    qw = q[:, :, -_WINDOW:, :].reshape(BH, _WINDOW, D)
    k3 = k.reshape(BH, S, D)
    return pl.pallas_call(
        _pooled_body,
        grid=(BH,),
        in_specs=[pl.BlockSpec((1, _WINDOW, D), lambda h: (h, 0, 0)),
                  pl.BlockSpec((1, S, D), lambda h: (h, 0, 0))],
        out_specs=pl.BlockSpec((1, 1, S), lambda h: (h, 0, 0)),
        out_shape=jax.ShapeDtypeStruct((BH, 1, S), jnp.float32),
    )(qw, k3).reshape(BH, S)


def _rank_body(vrow_ref, ranks_ref):
    S = vrow_ref.shape[2]
    vrow = vrow_ref[0]
    vcol = jnp.reshape(vrow, (S, 1))
    jidx = lax.broadcasted_iota(jnp.int32, (S, 128), 0)
    lidx = lax.broadcasted_iota(jnp.int32, (S, 128), 1)
    jml = jidx - lidx

    def body(ic, _):
        vi = vrow_ref[0, 0:1, pl.ds(pl.multiple_of(ic * 128, 128), 128)]
        gt = vcol > vi
        eq = vcol == vi
        jlt = jml < ic * 128
        cnt = jnp.sum((gt | (eq & jlt)).astype(jnp.int32), axis=0,
                      keepdims=True)
        ranks_ref[0, 0:1, pl.ds(pl.multiple_of(ic * 128, 128), 128)] = cnt
        return 0

    lax.fori_loop(0, S // 128, body, 0)


def _ranks_call(pooled_pad):
    BH, S = pooled_pad.shape
    p3 = pooled_pad.reshape(BH, 1, S)
    out = pl.pallas_call(
        _rank_body,
        grid=(BH,),
        in_specs=[pl.BlockSpec((1, 1, S), lambda h: (h, 0, 0))],
        out_specs=pl.BlockSpec((1, 1, S), lambda h: (h, 0, 0)),
        out_shape=jax.ShapeDtypeStruct((BH, 1, S), jnp.int32),
    )(p3)
    return out.reshape(BH, S)


def _sc_gather(k2, v2, ranks, BH, S, D, topk):
    cap = topk + _WINDOW
    nchunk = cap // 128
    mesh = plsc.VectorSubcoreMesh(core_axis_name="c", subcore_axis_name="s")
    num_cores = mesh.num_cores

    @functools.partial(
        pl.kernel,
        out_type=[jax.ShapeDtypeStruct((BH * cap, D), jnp.float32),
                  jax.ShapeDtypeStruct((BH * cap, D), jnp.float32)],
        mesh=mesh,
        scratch_types=[
            pltpu.VMEM((S,), jnp.int32),
            pltpu.VMEM((cap,), jnp.int32),
            pltpu.VMEM((128, D), jnp.float32),
            pltpu.VMEM((128, D), jnp.float32),
            pltpu.SemaphoreType.DMA,
            pltpu.SemaphoreType.DMA,
        ],
        compiler_params=pltpu.CompilerParams(needs_layout_passes=False),
    )
    def sc_k(k2_hbm, v2_hbm, ranks_hbm, ko_hbm, vo_hbm,
             rank_v, idx1, bufk, bufv, semk, semv):
        w = lax.axis_index("s") * num_cores + lax.axis_index("c")
        pltpu.sync_copy(ranks_hbm.at[w], rank_v)

        def build(c, _):
            r = rank_v[pl.ds(c * 16, 16)]
            src = w * S + c * 16 + lax.iota(jnp.int32, 16)
            plsc.store_scatter(idx1, [r], src, mask=r < topk)
            return 0

        lax.fori_loop(0, (S - _WINDOW) // 16, build, 0)

        for t in range(_WINDOW // 16):
            idx1[pl.ds(topk + t * 16, 16)] = (
                w * S + (S - _WINDOW) + t * 16 + lax.iota(jnp.int32, 16))

        def chunk(c, _):
            ix = idx1.at[pl.ds(c * 128, 128)]
            ck = pltpu.async_copy(k2_hbm.at[ix], bufk, semk)
            cv = pltpu.async_copy(v2_hbm.at[ix], bufv, semv)
            ck.wait()
            pltpu.sync_copy(bufk, ko_hbm.at[pl.ds(w * cap + c * 128, 128)])
            cv.wait()
            pltpu.sync_copy(bufv, vo_hbm.at[pl.ds(w * cap + c * 128, 128)])
            return 0

        lax.fori_loop(0, nchunk, chunk, 0)

    return sc_k(k2, v2, ranks)


def kernel(q, k, v):
    B, H, S, D = q.shape
    BH = B * H
    topk = _CAP - _WINDOW
    pooled_pad = _pooled_call(q, k)
    ranks = _ranks_call(pooled_pad)
    k2 = k.reshape(BH * S, D)
    v2 = v.reshape(BH * S, D)
    ko, vo = _sc_gather(k2, v2, ranks, BH, S, D, topk)
    key_states = ko.reshape(B, H, _CAP, D)
    value_states = vo.reshape(B, H, _CAP, D)
    return (key_states, value_states)


# speedup vs baseline: 3.4242x; 2.9612x over previous
import functools
import math

import jax
import jax.numpy as jnp
from jax import lax
from jax.experimental import pallas as pl
from jax.experimental.pallas import tpu as pltpu
from jax.experimental.pallas import tpu_sc as plsc

_WINDOW = 64
_CAP = 4096
_KS = 13
_NEG = jnp.finfo(jnp.float32).min


def _pooled_prologue(q, k):
    B, H, S, D = q.shape
    q_win = q[:, :, -_WINDOW:, :]
    attn = jnp.einsum('bhqd,bhkd->bhqk', q_win, k) / math.sqrt(D)
    idx = jnp.arange(_WINDOW)
    causal = jnp.where(idx[:, None] >= idx[None, :], 0.0, _NEG)
    attn = attn.at[:, :, :, -_WINDOW:].add(causal[None, None, :, :])
    attn = jax.nn.softmax(attn.astype(jnp.float32), axis=-1)
    attn_sum = attn[:, :, :, : S - _WINDOW].sum(axis=-2)
    pad = _KS // 2
    pooled = jax.lax.reduce_window(
        attn_sum, 0.0, jax.lax.add,
        window_dimensions=(1, 1, _KS),
        window_strides=(1, 1, 1),
        padding=((0, 0), (0, 0), (pad, pad)),
    ) / float(_KS)
    return pooled


def _pooled_body(qw_ref, k_ref, out_ref):
    S = k_ref.shape[1]
    SP = S - _WINDOW
    qw = qw_ref[0]
    kk = k_ref[0]
    scores = lax.dot_general(qw, kk, (((1,), (1,)), ((), ())),
                             preferred_element_type=jnp.float32)
    scores = scores / math.sqrt(qw.shape[1])
    iq = lax.broadcasted_iota(jnp.int32, (_WINDOW, _WINDOW), 0)
    jw = lax.broadcasted_iota(jnp.int32, (_WINDOW, _WINDOW), 1)
    causal = jnp.where(iq >= jw, 0.0, _NEG)
    scores = jnp.concatenate([scores[:, :SP], scores[:, SP:] + causal], axis=1)
    m = jnp.max(scores, axis=1, keepdims=True)
    e = jnp.exp(scores - m)
    s = jnp.sum(e, axis=1, keepdims=True)
    p = e / s
    asum = jnp.sum(p[:, :SP], axis=0, keepdims=True)
    z = jnp.zeros((1, _KS // 2), jnp.float32)
    padded = jnp.concatenate([z, asum, z], axis=1)
    acc = padded[:, 0:SP]
    for u in range(1, _KS):
        acc = acc + padded[:, u:u + SP]
    pooled = acc / float(_KS)
    out_ref[0] = jnp.concatenate(
        [pooled, jnp.full((1, _WINDOW), -1.0, jnp.float32)], axis=1)


def _pooled_call(q, k):
    B, H, S, D = q.shape
    BH = B * H
    qw = q[:, :, -_WINDOW:, :].reshape(BH, _WINDOW, D)
    k3 = k.reshape(BH, S, D)
    return pl.pallas_call(
        _pooled_body,
        grid=(BH,),
        in_specs=[pl.BlockSpec((1, _WINDOW, D), lambda h: (h, 0, 0)),
                  pl.BlockSpec((1, S, D), lambda h: (h, 0, 0))],
        out_specs=pl.BlockSpec((1, 1, S), lambda h: (h, 0, 0)),
        out_shape=jax.ShapeDtypeStruct((BH, 1, S), jnp.float32),
    )(qw, k3).reshape(BH, S)


def _rank_body(vrow_ref, ranks_ref):
    S = vrow_ref.shape[2]
    vrow = vrow_ref[0]
    vcol = jnp.reshape(vrow, (S, 1))
    jd = lax.broadcasted_iota(jnp.int32, (128, 128), 0)
    ld = lax.broadcasted_iota(jnp.int32, (128, 128), 1)
    jlt_d = jd < ld

    def count(mf):
        ones = jnp.ones((1, mf.shape[0]), jnp.float32)
        return lax.dot_general(ones, mf, (((1,), (0,)), ((), ())),
                               preferred_element_type=jnp.float32)

    for ic in range(S // 128):
        lo, hi = ic * 128, (ic + 1) * 128
        vi = vrow[0:1, lo:hi]
        vd = vcol[lo:hi]
        gt = vd > vi
        eq = vd == vi
        cnt = count(jnp.where(gt | (eq & jlt_d), 1.0, 0.0))
        if lo > 0:
            cnt = cnt + count(jnp.where(vcol[:lo] >= vi, 1.0, 0.0))
        if hi < S:
            cnt = cnt + count(jnp.where(vcol[hi:] > vi, 1.0, 0.0))
        ranks_ref[0, 0:1, lo:hi] = cnt.astype(jnp.int32)


def _ranks_call(pooled_pad):
    BH, S = pooled_pad.shape
    p3 = pooled_pad.reshape(BH, 1, S)
    out = pl.pallas_call(
        _rank_body,
        grid=(BH,),
        in_specs=[pl.BlockSpec((1, 1, S), lambda h: (h, 0, 0))],
        out_specs=pl.BlockSpec((1, 1, S), lambda h: (h, 0, 0)),
        out_shape=jax.ShapeDtypeStruct((BH, 1, S), jnp.int32),
    )(p3)
    return out.reshape(BH, S)


def _sc_gather(k2, v2, ranks, BH, S, D, topk):
    cap = topk + _WINDOW
    nchunk = cap // 128
    mesh = plsc.VectorSubcoreMesh(core_axis_name="c", subcore_axis_name="s")
    num_cores = mesh.num_cores

    @functools.partial(
        pl.kernel,
        out_type=[jax.ShapeDtypeStruct((BH * cap, D), jnp.float32),
                  jax.ShapeDtypeStruct((BH * cap, D), jnp.float32)],
        mesh=mesh,
        scratch_types=[
            pltpu.VMEM((S,), jnp.int32),
            pltpu.VMEM((cap,), jnp.int32),
            pltpu.VMEM((128, D), jnp.float32),
            pltpu.VMEM((128, D), jnp.float32),
            pltpu.SemaphoreType.DMA,
            pltpu.SemaphoreType.DMA,
        ],
        compiler_params=pltpu.CompilerParams(needs_layout_passes=False),
    )
    def sc_k(k2_hbm, v2_hbm, ranks_hbm, ko_hbm, vo_hbm,
             rank_v, idx1, bufk, bufv, semk, semv):
        w = lax.axis_index("s") * num_cores + lax.axis_index("c")
        pltpu.sync_copy(ranks_hbm.at[w], rank_v)

        def build(c, _):
            r = rank_v[pl.ds(c * 16, 16)]
            src = w * S + c * 16 + lax.iota(jnp.int32, 16)
            plsc.store_scatter(idx1, [r], src, mask=r < topk)
            return 0

        lax.fori_loop(0, (S - _WINDOW) // 16, build, 0)

        for t in range(_WINDOW // 16):
            idx1[pl.ds(topk + t * 16, 16)] = (
                w * S + (S - _WINDOW) + t * 16 + lax.iota(jnp.int32, 16))

        def chunk(c, _):
            ix = idx1.at[pl.ds(c * 128, 128)]
            ck = pltpu.async_copy(k2_hbm.at[ix], bufk, semk)
            cv = pltpu.async_copy(v2_hbm.at[ix], bufv, semv)
            ck.wait()
            pltpu.sync_copy(bufk, ko_hbm.at[pl.ds(w * cap + c * 128, 128)])
            cv.wait()
            pltpu.sync_copy(bufv, vo_hbm.at[pl.ds(w * cap + c * 128, 128)])
            return 0

        lax.fori_loop(0, nchunk, chunk, 0)

    return sc_k(k2, v2, ranks)


def kernel(q, k, v):
    B, H, S, D = q.shape
    BH = B * H
    topk = _CAP - _WINDOW
    pooled_pad = _pooled_call(q, k)
    ranks = _ranks_call(pooled_pad)
    k2 = k.reshape(BH * S, D)
    v2 = v.reshape(BH * S, D)
    ko, vo = _sc_gather(k2, v2, ranks, BH, S, D, topk)
    key_states = ko.reshape(B, H, _CAP, D)
    value_states = vo.reshape(B, H, _CAP, D)
    return (key_states, value_states)


# speedup vs baseline: 3.4869x; 1.0183x over previous
import functools
import math

import jax
import jax.numpy as jnp
from jax import lax
from jax.experimental import pallas as pl
from jax.experimental.pallas import tpu as pltpu
from jax.experimental.pallas import tpu_sc as plsc

_WINDOW = 64
_CAP = 4096
_KS = 13
_NEG = jnp.finfo(jnp.float32).min


def _pooled_prologue(q, k):
    B, H, S, D = q.shape
    q_win = q[:, :, -_WINDOW:, :]
    attn = jnp.einsum('bhqd,bhkd->bhqk', q_win, k) / math.sqrt(D)
    idx = jnp.arange(_WINDOW)
    causal = jnp.where(idx[:, None] >= idx[None, :], 0.0, _NEG)
    attn = attn.at[:, :, :, -_WINDOW:].add(causal[None, None, :, :])
    attn = jax.nn.softmax(attn.astype(jnp.float32), axis=-1)
    attn_sum = attn[:, :, :, : S - _WINDOW].sum(axis=-2)
    pad = _KS // 2
    pooled = jax.lax.reduce_window(
        attn_sum, 0.0, jax.lax.add,
        window_dimensions=(1, 1, _KS),
        window_strides=(1, 1, 1),
        padding=((0, 0), (0, 0), (pad, pad)),
    ) / float(_KS)
    return pooled


def _score_rank_body(qw_ref, k_ref, ranks_ref):
    S = k_ref.shape[1]
    SP = S - _WINDOW
    qw = qw_ref[0]
    kk = k_ref[0]
    scores = lax.dot_general(qw, kk, (((1,), (1,)), ((), ())),
                             preferred_element_type=jnp.float32)
    scores = scores / math.sqrt(qw.shape[1])
    iq = lax.broadcasted_iota(jnp.int32, (_WINDOW, _WINDOW), 0)
    jw = lax.broadcasted_iota(jnp.int32, (_WINDOW, _WINDOW), 1)
    causal = jnp.where(iq >= jw, 0.0, _NEG)
    scores = jnp.concatenate([scores[:, :SP], scores[:, SP:] + causal], axis=1)
    m = jnp.max(scores, axis=1, keepdims=True)
    e = jnp.exp(scores - m)
    s = jnp.sum(e, axis=1, keepdims=True)
    p = e / s
    asum = jnp.sum(p[:, :SP], axis=0, keepdims=True)
    z = jnp.zeros((1, _KS // 2), jnp.float32)
    padded = jnp.concatenate([z, asum, z], axis=1)
    acc = padded[:, 0:SP]
    for u in range(1, _KS):
        acc = acc + padded[:, u:u + SP]
    pooled = acc / float(_KS)
    vrow = jnp.concatenate(
        [pooled, jnp.full((1, _WINDOW), -1.0, jnp.float32)], axis=1)
    vcol = jnp.reshape(vrow, (S, 1))
    jd = lax.broadcasted_iota(jnp.int32, (128, 128), 0)
    ld = lax.broadcasted_iota(jnp.int32, (128, 128), 1)
    jlt_d = jd < ld

    def count(mf):
        ones = jnp.ones((1, mf.shape[0]), jnp.float32)
        return lax.dot_general(ones, mf, (((1,), (0,)), ((), ())),
                               preferred_element_type=jnp.float32,
                               precision=lax.Precision.DEFAULT)

    for ic in range(S // 128):
        lo, hi = ic * 128, (ic + 1) * 128
        vi = vrow[0:1, lo:hi]
        vd = vcol[lo:hi]
        gt = vd > vi
        eq = vd == vi
        cnt = count(jnp.where(gt | (eq & jlt_d), 1.0, 0.0))
        if lo > 0:
            cnt = cnt + count(jnp.where(vcol[:lo] >= vi, 1.0, 0.0))
        if hi < S:
            cnt = cnt + count(jnp.where(vcol[hi:] > vi, 1.0, 0.0))
        ranks_ref[0, 0:1, lo:hi] = cnt.astype(jnp.int32)


def _ranks_call(q, k):
    B, H, S, D = q.shape
    BH = B * H
    qw = q[:, :, -_WINDOW:, :].reshape(BH, _WINDOW, D)
    k3 = k.reshape(BH, S, D)
    out = pl.pallas_call(
        _score_rank_body,
        grid=(BH,),
        in_specs=[pl.BlockSpec((1, _WINDOW, D), lambda h: (h, 0, 0)),
                  pl.BlockSpec((1, S, D), lambda h: (h, 0, 0))],
        out_specs=pl.BlockSpec((1, 1, S), lambda h: (h, 0, 0)),
        out_shape=jax.ShapeDtypeStruct((BH, 1, S), jnp.int32),
    )(qw, k3)
    return out.reshape(BH, S)


def _sc_gather(k2, v2, ranks, BH, S, D, topk):
    cap = topk + _WINDOW
    nchunk = cap // 128
    mesh = plsc.VectorSubcoreMesh(core_axis_name="c", subcore_axis_name="s")
    num_cores = mesh.num_cores

    @functools.partial(
        pl.kernel,
        out_type=[jax.ShapeDtypeStruct((BH * cap, D), jnp.float32),
                  jax.ShapeDtypeStruct((BH * cap, D), jnp.float32)],
        mesh=mesh,
        scratch_types=[
            pltpu.VMEM((S,), jnp.int32),
            pltpu.VMEM((cap,), jnp.int32),
            pltpu.VMEM((128, D), jnp.float32),
            pltpu.VMEM((128, D), jnp.float32),
            pltpu.VMEM((128, D), jnp.float32),
            pltpu.VMEM((128, D), jnp.float32),
            [pltpu.SemaphoreType.DMA] * 8,
        ],
        compiler_params=pltpu.CompilerParams(needs_layout_passes=False),
    )
    def sc_k(k2_hbm, v2_hbm, ranks_hbm, ko_hbm, vo_hbm,
             rank_v, idx1, bufka, bufva, bufkb, bufvb, sems):
        w = lax.axis_index("s") * num_cores + lax.axis_index("c")
        pltpu.sync_copy(ranks_hbm.at[w], rank_v)

        def build(c, _):
            r = rank_v[pl.ds(c * 16, 16)]
            src = w * S + c * 16 + lax.iota(jnp.int32, 16)
            plsc.store_scatter(idx1, [r], src, mask=r < topk)
            return 0

        lax.fori_loop(0, (S - _WINDOW) // 16, build, 0)

        for t in range(_WINDOW // 16):
            idx1[pl.ds(topk + t * 16, 16)] = (
                w * S + (S - _WINDOW) + t * 16 + lax.iota(jnp.int32, 16))

        def chunk2(c2, _):
            a, b = c2 * 2, c2 * 2 + 1
            ixa = idx1.at[pl.ds(a * 128, 128)]
            ixb = idx1.at[pl.ds(b * 128, 128)]
            oa = pl.ds(w * cap + a * 128, 128)
            ob = pl.ds(w * cap + b * 128, 128)
            gka = pltpu.async_copy(k2_hbm.at[ixa], bufka, sems[0])
            gva = pltpu.async_copy(v2_hbm.at[ixa], bufva, sems[1])
            gkb = pltpu.async_copy(k2_hbm.at[ixb], bufkb, sems[2])
            gvb = pltpu.async_copy(v2_hbm.at[ixb], bufvb, sems[3])
            gka.wait()
            wka = pltpu.async_copy(bufka, ko_hbm.at[oa], sems[4])
            gva.wait()
            wva = pltpu.async_copy(bufva, vo_hbm.at[oa], sems[5])
            gkb.wait()
            wkb = pltpu.async_copy(bufkb, ko_hbm.at[ob], sems[6])
            gvb.wait()
            wvb = pltpu.async_copy(bufvb, vo_hbm.at[ob], sems[7])
            wka.wait()
            wva.wait()
            wkb.wait()
            wvb.wait()
            return 0

        lax.fori_loop(0, nchunk // 2, chunk2, 0)

    return sc_k(k2, v2, ranks)


def kernel(q, k, v):
    B, H, S, D = q.shape
    BH = B * H
    topk = _CAP - _WINDOW
    ranks = _ranks_call(q, k)
    k2 = k.reshape(BH * S, D)
    v2 = v.reshape(BH * S, D)
    ko, vo = _sc_gather(k2, v2, ranks, BH, S, D, topk)
    key_states = ko.reshape(B, H, _CAP, D)
    value_states = vo.reshape(B, H, _CAP, D)
    return (key_states, value_states)


# speedup vs baseline: 3.4896x; 1.0008x over previous
import functools
import math

import jax
import jax.numpy as jnp
from jax import lax
from jax.experimental import pallas as pl
from jax.experimental.pallas import tpu as pltpu
from jax.experimental.pallas import tpu_sc as plsc

_WINDOW = 64
_CAP = 4096
_KS = 13
_NEG = jnp.finfo(jnp.float32).min


def _score_rank_body(qw_ref, k_ref, ranks_ref):
    S = k_ref.shape[1]
    SP = S - _WINDOW
    qw = qw_ref[0]
    kk = k_ref[0]
    scores = lax.dot_general(qw, kk, (((1,), (1,)), ((), ())),
                             preferred_element_type=jnp.float32)
    scores = scores / math.sqrt(qw.shape[1])
    iq = lax.broadcasted_iota(jnp.int32, (_WINDOW, _WINDOW), 0)
    jw = lax.broadcasted_iota(jnp.int32, (_WINDOW, _WINDOW), 1)
    causal = jnp.where(iq >= jw, 0.0, _NEG)
    scores = jnp.concatenate([scores[:, :SP], scores[:, SP:] + causal], axis=1)
    m = jnp.max(scores, axis=1, keepdims=True)
    e = jnp.exp(scores - m)
    s = jnp.sum(e, axis=1, keepdims=True)
    p = e / s
    asum = jnp.sum(p[:, :SP], axis=0, keepdims=True)
    z = jnp.zeros((1, _KS // 2), jnp.float32)
    padded = jnp.concatenate([z, asum, z], axis=1)
    acc = padded[:, 0:SP]
    for u in range(1, _KS):
        acc = acc + padded[:, u:u + SP]
    pooled = acc / float(_KS)
    vrow = jnp.concatenate(
        [pooled, jnp.full((1, _WINDOW), -1.0, jnp.float32)], axis=1)
    vcol = jnp.reshape(vrow, (S, 1))
    jd = lax.broadcasted_iota(jnp.int32, (128, 128), 0)
    ld = lax.broadcasted_iota(jnp.int32, (128, 128), 1)
    jlt_d = jd < ld

    def count(mf):
        ones = jnp.ones((1, mf.shape[0]), jnp.float32)
        return lax.dot_general(ones, mf, (((1,), (0,)), ((), ())),
                               preferred_element_type=jnp.float32,
                               precision=lax.Precision.DEFAULT)

    for ic in range(S // 128):
        lo, hi = ic * 128, (ic + 1) * 128
        vi = vrow[0:1, lo:hi]
        vd = vcol[lo:hi]
        gt = vd > vi
        eq = vd == vi
        cnt = count(jnp.where(gt | (eq & jlt_d), 1.0, 0.0))
        if lo > 0:
            cnt = cnt + count(jnp.where(vcol[:lo] >= vi, 1.0, 0.0))
        if hi < S:
            cnt = cnt + count(jnp.where(vcol[hi:] > vi, 1.0, 0.0))
        ranks_ref[0, 0:1, lo:hi] = cnt.astype(jnp.int32)


def _ranks_call(q, k):
    B, H, S, D = q.shape
    BH = B * H
    qw = q[:, :, -_WINDOW:, :].reshape(BH, _WINDOW, D)
    k3 = k.reshape(BH, S, D)
    out = pl.pallas_call(
        _score_rank_body,
        grid=(BH,),
        in_specs=[pl.BlockSpec((1, _WINDOW, D), lambda h: (h, 0, 0)),
                  pl.BlockSpec((1, S, D), lambda h: (h, 0, 0))],
        out_specs=pl.BlockSpec((1, 1, S), lambda h: (h, 0, 0)),
        out_shape=jax.ShapeDtypeStruct((BH, 1, S), jnp.int32),
    )(qw, k3)
    return out.reshape(BH, S)


def _sc_gather(k2, v2, ranks, BH, S, D, topk):
    cap = topk + _WINDOW
    nchunk = cap // 128
    mesh = plsc.VectorSubcoreMesh(core_axis_name="c", subcore_axis_name="s")
    num_cores = mesh.num_cores

    @functools.partial(
        pl.kernel,
        out_type=[jax.ShapeDtypeStruct((BH * cap, D), jnp.float32),
                  jax.ShapeDtypeStruct((BH * cap, D), jnp.float32)],
        mesh=mesh,
        scratch_types=[
            pltpu.VMEM((S,), jnp.int32),
            pltpu.VMEM((cap,), jnp.int32),
            pltpu.VMEM((128, D), jnp.float32),
            pltpu.VMEM((128, D), jnp.float32),
            pltpu.VMEM((128, D), jnp.float32),
            pltpu.VMEM((128, D), jnp.float32),
            [pltpu.SemaphoreType.DMA] * 8,
        ],
        compiler_params=pltpu.CompilerParams(needs_layout_passes=False),
    )
    def sc_k(k2_hbm, v2_hbm, ranks_hbm, ko_hbm, vo_hbm,
             rank_v, idx1, bufka, bufva, bufkb, bufvb, sems):
        w = lax.axis_index("s") * num_cores + lax.axis_index("c")
        pltpu.sync_copy(ranks_hbm.at[w], rank_v)

        def build(c, _):
            r = rank_v[pl.ds(c * 16, 16)]
            src = w * S + c * 16 + lax.iota(jnp.int32, 16)
            plsc.store_scatter(idx1, [r], src, mask=r < topk)
            return 0

        lax.fori_loop(0, (S - _WINDOW) // 16, build, 0)

        for t in range(_WINDOW // 16):
            idx1[pl.ds(topk + t * 16, 16)] = (
                w * S + (S - _WINDOW) + t * 16 + lax.iota(jnp.int32, 16))

        def chunk2(c2, _):
            a, b = c2 * 2, c2 * 2 + 1
            ixa = idx1.at[pl.ds(a * 128, 128)]
            ixb = idx1.at[pl.ds(b * 128, 128)]
            oa = pl.ds(w * cap + a * 128, 128)
            ob = pl.ds(w * cap + b * 128, 128)
            gka = pltpu.async_copy(k2_hbm.at[ixa], bufka, sems[0])
            gva = pltpu.async_copy(v2_hbm.at[ixa], bufva, sems[1])
            gkb = pltpu.async_copy(k2_hbm.at[ixb], bufkb, sems[2])
            gvb = pltpu.async_copy(v2_hbm.at[ixb], bufvb, sems[3])
            gka.wait()
            wka = pltpu.async_copy(bufka, ko_hbm.at[oa], sems[4])
            gva.wait()
            wva = pltpu.async_copy(bufva, vo_hbm.at[oa], sems[5])
            gkb.wait()
            wkb = pltpu.async_copy(bufkb, ko_hbm.at[ob], sems[6])
            gvb.wait()
            wvb = pltpu.async_copy(bufvb, vo_hbm.at[ob], sems[7])
            wka.wait()
            wva.wait()
            wkb.wait()
            wvb.wait()
            return 0

        lax.fori_loop(0, nchunk // 2, chunk2, 0)

    return sc_k(k2, v2, ranks)


def kernel(q, k, v):
    B, H, S, D = q.shape
    BH = B * H
    topk = _CAP - _WINDOW
    ranks = _ranks_call(q, k)
    k2 = k.reshape(BH * S, D)
    v2 = v.reshape(BH * S, D)
    ko, vo = _sc_gather(k2, v2, ranks, BH, S, D, topk)
    key_states = ko.reshape(B, H, _CAP, D)
    value_states = vo.reshape(B, H, _CAP, D)
    return (key_states, value_states)
